# initial kernel scaffold (unmeasured)
import jax
import jax.numpy as jnp
from jax import lax
from jax.experimental import pallas as pl
from jax.experimental.pallas import tpu as pltpu

N_DEV = 4
SQ = 2048
SKV = 2048
D_MODEL = 1024
HQ_TOTAL = 32
HQ_PER = 8
DH = 128
BLK = 64
SCALE = 0.08838834764831843


def _ag_body(w_ref, out_ref, send_sems, recv_sems):
    my = lax.axis_index("i")
    left = lax.rem(my + N_DEV - 1, N_DEV)
    right = lax.rem(my + 1, N_DEV)

    barrier = pltpu.get_barrier_semaphore()
    for nbr in (left, right):
        pl.semaphore_signal(
            barrier, inc=1, device_id=(nbr,), device_id_type=pl.DeviceIdType.MESH
        )
    pl.semaphore_wait(barrier, 2)

    out_ref[pl.ds(my, 1)] = w_ref[...][None]

    for h in range(N_DEV - 1):
        origin = lax.rem(my - h + N_DEV, N_DEV)
        rdma = pltpu.make_async_remote_copy(
            src_ref=out_ref.at[origin],
            dst_ref=out_ref.at[origin],
            send_sem=send_sems.at[h],
            recv_sem=recv_sems.at[h],
            device_id=(right,),
            device_id_type=pl.DeviceIdType.MESH,
        )
        rdma.start()
        rdma.wait()


def _attn_body(x_ref, wq_ref, wo_ref, k_ref, v_ref, out_ref):
    h = pl.program_id(0)
    x = x_ref[...]
    wq = wq_ref[0, 0]
    wo = wo_ref[0, 0]
    k = k_ref[:, 0, :]
    v = v_ref[:, 0, :]

    q = jnp.dot(x, wq, preferred_element_type=jnp.float32)
    s = lax.dot_general(
        q.astype(jnp.bfloat16),
        k,
        (((1,), (1,)), ((), ())),
        preferred_element_type=jnp.float32,
    ) * SCALE

    qb = lax.broadcasted_iota(jnp.int32, (SQ, SKV), 0) // BLK
    kb = lax.broadcasted_iota(jnp.int32, (SQ, SKV), 1) // BLK
    mask = (qb == kb) | (kb == 0) | (lax.rem(qb + kb, 3) == 0)
    s = jnp.where(mask, s, -1e9)

    mx = jnp.max(s, axis=1, keepdims=True)
    e = jnp.exp(s - mx)
    d = jnp.sum(e, axis=1, keepdims=True)
    ctx = lax.dot_general(
        e.astype(jnp.bfloat16),
        v,
        (((1,), (0,)), ((), ())),
        preferred_element_type=jnp.float32,
    )
    ctx = ctx / d
    contrib = jnp.dot(
        ctx.astype(jnp.bfloat16), wo, preferred_element_type=jnp.float32
    )

    @pl.when(h == 0)
    def _():
        out_ref[...] = contrib

    @pl.when(h != 0)
    def _():
        out_ref[...] += contrib


def kernel(x, Wq, K_ext, V_ext, Wo):
    my = lax.axis_index("i")
    xb = x[0].astype(jnp.bfloat16)
    kb = lax.dynamic_index_in_dim(K_ext, my, 0, keepdims=False).astype(
        jnp.bfloat16
    )
    vb = lax.dynamic_index_in_dim(V_ext, my, 0, keepdims=False).astype(
        jnp.bfloat16
    )
    w = jnp.stack([Wq.astype(jnp.bfloat16), Wo.astype(jnp.bfloat16)])

    w_all = pl.pallas_call(
        _ag_body,
        out_shape=jax.ShapeDtypeStruct((N_DEV, 2, D_MODEL, D_MODEL), jnp.bfloat16),
        in_specs=[pl.BlockSpec(memory_space=pltpu.VMEM)],
        out_specs=pl.BlockSpec(memory_space=pltpu.VMEM),
        scratch_shapes=[
            pltpu.SemaphoreType.DMA((N_DEV - 1,)),
            pltpu.SemaphoreType.DMA((N_DEV - 1,)),
        ],
        compiler_params=pltpu.CompilerParams(collective_id=0),
    )(w)

    out = pl.pallas_call(
        _attn_body,
        grid=(HQ_TOTAL,),
        out_shape=jax.ShapeDtypeStruct((SQ, D_MODEL), jnp.float32),
        in_specs=[
            pl.BlockSpec((SQ, D_MODEL), lambda h: (0, 0)),
            pl.BlockSpec(
                (1, 1, D_MODEL, DH), lambda h: (h // HQ_PER, 0, 0, h % HQ_PER)
            ),
            pl.BlockSpec(
                (1, 1, DH, D_MODEL), lambda h: (h // HQ_PER, 1, h % HQ_PER, 0)
            ),
            pl.BlockSpec((SKV, 1, DH), lambda h: (0, h, 0)),
            pl.BlockSpec((SKV, 1, DH), lambda h: (0, h, 0)),
        ],
        out_specs=pl.BlockSpec((SQ, D_MODEL), lambda h: (0, 0)),
        compiler_params=pltpu.CompilerParams(
            dimension_semantics=("arbitrary",),
        ),
    )(xb, w_all, w_all, kb, vb)

    return out[None]


# baseline (device time: 732830 ns/iter reference)
import jax
import jax.numpy as jnp
from jax import lax
from jax.experimental import pallas as pl
from jax.experimental.pallas import tpu as pltpu

N_DEV = 4
SQ = 2048
SKV = 2048
D_MODEL = 1024
HQ_TOTAL = 32
HQ_PER = 8
DH = 128
BLK = 64
SCALE = 0.08838834764831843


def _ag_body(w_ref, out_ref, send_sems, recv_sems):
    my = lax.axis_index("i")
    left = lax.rem(my + N_DEV - 1, N_DEV)
    right = lax.rem(my + 1, N_DEV)

    barrier = pltpu.get_barrier_semaphore()
    for nbr in (left, right):
        pl.semaphore_signal(
            barrier, inc=1, device_id=(nbr,), device_id_type=pl.DeviceIdType.MESH
        )
    pl.semaphore_wait(barrier, 2)

    out_ref[pl.ds(my, 1)] = w_ref[...][None]

    for h in range(N_DEV - 1):
        origin = lax.rem(my - h + N_DEV, N_DEV)
        rdma = pltpu.make_async_remote_copy(
            src_ref=out_ref.at[origin],
            dst_ref=out_ref.at[origin],
            send_sem=send_sems.at[h],
            recv_sem=recv_sems.at[h],
            device_id=(right,),
            device_id_type=pl.DeviceIdType.MESH,
        )
        rdma.start()
        rdma.wait()


def _attn_body(x_ref, wq_ref, wo_ref, k_ref, v_ref, out_ref):
    h = pl.program_id(0)
    x = x_ref[...]
    wq = wq_ref[0, 0]
    wo = wo_ref[0, 0]
    k = k_ref[0]
    v = v_ref[0]

    q = jnp.dot(x, wq, preferred_element_type=jnp.float32)
    s = lax.dot_general(
        q.astype(jnp.bfloat16),
        k,
        (((1,), (1,)), ((), ())),
        preferred_element_type=jnp.float32,
    ) * SCALE

    qb = lax.broadcasted_iota(jnp.int32, (SQ, SKV), 0) // BLK
    kb = lax.broadcasted_iota(jnp.int32, (SQ, SKV), 1) // BLK
    mask = (qb == kb) | (kb == 0) | (lax.rem(qb + kb, 3) == 0)
    s = jnp.where(mask, s, -1e9)

    mx = jnp.max(s, axis=1, keepdims=True)
    e = jnp.exp(s - mx)
    d = jnp.sum(e, axis=1, keepdims=True)
    ctx = lax.dot_general(
        e.astype(jnp.bfloat16),
        v,
        (((1,), (0,)), ((), ())),
        preferred_element_type=jnp.float32,
    )
    ctx = ctx / d
    contrib = jnp.dot(
        ctx.astype(jnp.bfloat16), wo, preferred_element_type=jnp.float32
    )

    @pl.when(h == 0)
    def _():
        out_ref[...] = contrib

    @pl.when(h != 0)
    def _():
        out_ref[...] += contrib


def kernel(x, Wq, K_ext, V_ext, Wo):
    my = lax.axis_index("i")
    xb = x[0].astype(jnp.bfloat16)
    kb = (
        lax.dynamic_index_in_dim(K_ext, my, 0, keepdims=False)
        .astype(jnp.bfloat16)
        .transpose(1, 0, 2)
    )
    vb = (
        lax.dynamic_index_in_dim(V_ext, my, 0, keepdims=False)
        .astype(jnp.bfloat16)
        .transpose(1, 0, 2)
    )
    w = jnp.stack([Wq.astype(jnp.bfloat16), Wo.astype(jnp.bfloat16)])

    w_all = pl.pallas_call(
        _ag_body,
        out_shape=jax.ShapeDtypeStruct((N_DEV, 2, D_MODEL, D_MODEL), jnp.bfloat16),
        in_specs=[pl.BlockSpec(memory_space=pltpu.VMEM)],
        out_specs=pl.BlockSpec(memory_space=pltpu.VMEM),
        scratch_shapes=[
            pltpu.SemaphoreType.DMA((N_DEV - 1,)),
            pltpu.SemaphoreType.DMA((N_DEV - 1,)),
        ],
        compiler_params=pltpu.CompilerParams(collective_id=0),
    )(w)

    out = pl.pallas_call(
        _attn_body,
        grid=(HQ_TOTAL,),
        out_shape=jax.ShapeDtypeStruct((SQ, D_MODEL), jnp.float32),
        in_specs=[
            pl.BlockSpec((SQ, D_MODEL), lambda h: (0, 0)),
            pl.BlockSpec(
                (1, 1, D_MODEL, DH), lambda h: (h // HQ_PER, 0, 0, h % HQ_PER)
            ),
            pl.BlockSpec(
                (1, 1, DH, D_MODEL), lambda h: (h // HQ_PER, 1, h % HQ_PER, 0)
            ),
            pl.BlockSpec((1, SKV, DH), lambda h: (h, 0, 0)),
            pl.BlockSpec((1, SKV, DH), lambda h: (h, 0, 0)),
        ],
        out_specs=pl.BlockSpec((SQ, D_MODEL), lambda h: (0, 0)),
        compiler_params=pltpu.CompilerParams(
            dimension_semantics=("arbitrary",),
        ),
    )(xb, w_all, w_all, kb, vb)

    return out[None]


# device time: 581999 ns/iter; 1.2592x vs baseline; 1.2592x over previous
import jax
import jax.numpy as jnp
from jax import lax
from jax.experimental import pallas as pl
from jax.experimental.pallas import tpu as pltpu

N_DEV = 4
SQ = 2048
SKV = 2048
D_MODEL = 1024
HQ_TOTAL = 32
HQ_PER = 8
DH = 128
BLK = 64
SCALE = 0.08838834764831843


def _ag_body(w_ref, out_ref, send_sems, recv_sems):
    my = lax.axis_index("i")
    left = lax.rem(my + N_DEV - 1, N_DEV)
    right = lax.rem(my + 1, N_DEV)

    barrier = pltpu.get_barrier_semaphore()
    for nbr in (left, right):
        pl.semaphore_signal(
            barrier, inc=1, device_id=(nbr,), device_id_type=pl.DeviceIdType.MESH
        )
    pl.semaphore_wait(barrier, 2)

    out_ref[pl.ds(my, 1)] = w_ref[...][None]

    for h in range(N_DEV - 1):
        origin = lax.rem(my - h + N_DEV, N_DEV)
        rdma = pltpu.make_async_remote_copy(
            src_ref=out_ref.at[origin],
            dst_ref=out_ref.at[origin],
            send_sem=send_sems.at[h],
            recv_sem=recv_sems.at[h],
            device_id=(right,),
            device_id_type=pl.DeviceIdType.MESH,
        )
        rdma.start()
        rdma.wait()


SQB = 512
N_QB = SQ // SQB


def _attn_body(x_ref, wq_ref, wo_ref, k_ref, v_ref, out_ref, bias_ref):
    h = pl.program_id(0)
    qb = pl.program_id(1)

    @pl.when((h == 0) & (qb == 0))
    def _():
        qi = lax.broadcasted_iota(jnp.int32, (SQ, SKV), 0) // BLK
        ki = lax.broadcasted_iota(jnp.int32, (SQ, SKV), 1) // BLK
        keep = (qi == ki) | (ki == 0) | (lax.rem(qi + ki, 3) == 0)
        bias_ref[...] = jnp.where(keep, 0.0, -1e9).astype(jnp.bfloat16)

    rows = pl.ds(qb * SQB, SQB)
    x = x_ref[rows, :]
    wq = wq_ref[0, 0]
    wo = wo_ref[0, 0]
    k = k_ref[0]
    v = v_ref[0]

    q = jnp.dot(x, wq, preferred_element_type=jnp.float32)
    qs = (q * SCALE).astype(jnp.bfloat16)
    s = lax.dot_general(
        qs,
        k,
        (((1,), (1,)), ((), ())),
        preferred_element_type=jnp.float32,
    )
    e = jnp.exp(s + bias_ref[rows, :])
    eb = e.astype(jnp.bfloat16)
    ctx = lax.dot_general(
        eb,
        v,
        (((1,), (0,)), ((), ())),
        preferred_element_type=jnp.float32,
    )
    d = jnp.sum(e, axis=1, keepdims=True)
    contrib = jnp.dot(
        (ctx / d).astype(jnp.bfloat16), wo, preferred_element_type=jnp.float32
    )

    @pl.when(h == 0)
    def _():
        out_ref[rows, :] = contrib

    @pl.when(h != 0)
    def _():
        out_ref[rows, :] += contrib


def kernel(x, Wq, K_ext, V_ext, Wo):
    my = lax.axis_index("i")
    xb = x[0].astype(jnp.bfloat16)
    kb = (
        lax.dynamic_index_in_dim(K_ext, my, 0, keepdims=False)
        .astype(jnp.bfloat16)
        .transpose(1, 0, 2)
    )
    vb = (
        lax.dynamic_index_in_dim(V_ext, my, 0, keepdims=False)
        .astype(jnp.bfloat16)
        .transpose(1, 0, 2)
    )
    w = jnp.stack([Wq.astype(jnp.bfloat16), Wo.astype(jnp.bfloat16)])

    w_all = pl.pallas_call(
        _ag_body,
        out_shape=jax.ShapeDtypeStruct((N_DEV, 2, D_MODEL, D_MODEL), jnp.bfloat16),
        in_specs=[pl.BlockSpec(memory_space=pltpu.VMEM)],
        out_specs=pl.BlockSpec(memory_space=pltpu.VMEM),
        scratch_shapes=[
            pltpu.SemaphoreType.DMA((N_DEV - 1,)),
            pltpu.SemaphoreType.DMA((N_DEV - 1,)),
        ],
        compiler_params=pltpu.CompilerParams(collective_id=0),
    )(w)

    out = pl.pallas_call(
        _attn_body,
        grid=(HQ_TOTAL, N_QB),
        out_shape=jax.ShapeDtypeStruct((SQ, D_MODEL), jnp.float32),
        in_specs=[
            pl.BlockSpec((SQ, D_MODEL), lambda h, q: (0, 0)),
            pl.BlockSpec(
                (1, 1, D_MODEL, DH), lambda h, q: (h // HQ_PER, 0, 0, h % HQ_PER)
            ),
            pl.BlockSpec(
                (1, 1, DH, D_MODEL), lambda h, q: (h // HQ_PER, 1, h % HQ_PER, 0)
            ),
            pl.BlockSpec((1, SKV, DH), lambda h, q: (h, 0, 0)),
            pl.BlockSpec((1, SKV, DH), lambda h, q: (h, 0, 0)),
        ],
        out_specs=pl.BlockSpec((SQ, D_MODEL), lambda h, q: (0, 0)),
        scratch_shapes=[
            pltpu.VMEM((SQ, SKV), jnp.bfloat16),
        ],
        compiler_params=pltpu.CompilerParams(
            dimension_semantics=("arbitrary", "arbitrary"),
            vmem_limit_bytes=60 * 1024 * 1024,
        ),
    )(xb, w_all, w_all, kb, vb)

    return out[None]
